# baseline (speedup 1.0000x reference)
import functools

import jax
import jax.numpy as jnp
from jax import lax
from jax.experimental import pallas as pl
from jax.experimental.pallas import tpu as pltpu
from jax.experimental.pallas import tpu_sc as plsc

N = 50000
E = 800000
G = 64
D = 64
DH = 32
NC = 2
NS = 16
BE = 125
NROWS = E // BE
RPT = NROWS // NS
RPT1 = NROWS // (NC * NS)
NPT = N // NS

_mesh_cache = []


def _mesh():
  if not _mesh_cache:
    _mesh_cache.append(plsc.VectorSubcoreMesh(
        core_axis_name="c", subcore_axis_name="s",
        num_cores=NC, num_subcores=NS))
  return _mesh_cache[0]


IDXB = 20
RING = 5
PD = 3
NIB = RPT // IDXB
NIB1 = RPT1 // IDXB


def _agg_pipeline(tbl, srcm, dstm, out, scratch, dw, base, dbase, nib, s, c):
  (is0, is1, id0, id1, r0, r1, r2, r3, r4, acc,
   gs0, gs1, gs2, gs3, gs4, ss0, ss1, ss2, ss3, ss4, isem) = scratch
  iss = (is0, is1)
  ids = (id0, id1)
  ring = (r0, r1, r2, r3, r4)
  gsem = (gs0, gs1, gs2, gs3, gs4)
  ssem = (ss0, ss1, ss2, ss3, ss4)

  def zb(i, carry):
    ring[0][i, pl.ds(0, 16)] = jnp.zeros((16,), jnp.float32)
    if dw > 16:
      ring[0][i, pl.ds(16, 16)] = jnp.zeros((16,), jnp.float32)
    return carry
  lax.fori_loop(0, BE, zb, 0)

  def zc(k, carry):
    pltpu.sync_copy(ring[0], acc.at[pl.ds(s * NPT + k * BE, BE)])
    return carry
  lax.fori_loop(0, NPT // BE, zc, 0)
  plsc.subcore_barrier()

  def idx_load(nb, b, sync):
    sc_ = pltpu.async_copy(srcm.at[pl.ds(base + nb * IDXB, IDXB)],
                           iss[b], isem)
    dc_ = pltpu.async_copy(dstm.at[pl.ds(dbase + nb * IDXB, IDXB)],
                           ids[b], isem)
    if sync:
      sc_.wait()
      dc_.wait()

  def idx_wait(nb, b):
    pltpu.make_async_copy(srcm.at[pl.ds(0, IDXB)], iss[b], isem).wait()
    pltpu.make_async_copy(dstm.at[pl.ds(0, IDXB)], ids[b], isem).wait()

  def fire(b, q, r):
    pltpu.async_copy(tbl.at[iss[b].at[q]], ring[r], gsem[r])

  def drain(r):
    pltpu.make_async_copy(tbl.at[pl.ds(0, BE)], ring[r], gsem[r]).wait()

  def sdrain(r):
    pltpu.make_async_copy(ring[r], acc.at[pl.ds(0, BE)], ssem[r]).wait()

  idx_load(0, 0, True)

  def outer(nb0, carry):
    for b in range(2):
      nb = nb0 * 2 + b

      @pl.when(nb + 1 < nib)
      def _():
        idx_load(nb + 1, 1 - b, False)
      for q in range(PD):
        @pl.when(nb > 0)
        def _(q=q):
          sdrain(q % RING)
        fire(b, q, q % RING)
      for q in range(IDXB):
        r = q % RING
        drain(r)
        pltpu.async_copy(ring[r], acc.at[ids[b].at[q]], ssem[r], add=True)
        if q + PD < IDXB:
          rp = (q + PD) % RING
          if q < RING - PD:
            @pl.when(nb > 0)
            def _(rp=rp):
              sdrain(rp)
          else:
            sdrain(rp)
          fire(b, q + PD, rp)

      @pl.when(nb + 1 < nib)
      def _():
        idx_wait(nb + 1, 1 - b)
    return carry
  lax.fori_loop(0, nib // 2, outer, 0)
  for r in range(RING):
    sdrain(r)
  plsc.subcore_barrier()

  pltpu.sync_copy(acc.at[pl.ds(s * NPT, NPT)],
                  out.at[pl.ds(c * N + s * NPT, NPT)])


def _sc_agg_body(tbl, srcm, dstm, out, *scratch):
  c = lax.axis_index("c")
  s = lax.axis_index("s")
  _agg_pipeline(tbl, srcm, dstm, out, scratch, DH,
                c * NROWS + s * RPT, s * RPT, NIB, s, c)


def _sc_scratch(dw):
  return ([pltpu.VMEM((IDXB, BE), jnp.int32)] * 4
          + [pltpu.VMEM((BE, dw), jnp.float32)] * RING
          + [pltpu.VMEM_SHARED((N, dw), jnp.float32)]
          + [pltpu.SemaphoreType.DMA] * (2 * RING + 1))


def _sc_agg(h, srcm, dstm):
  return pl.kernel(
      _sc_agg_body,
      out_type=jax.ShapeDtypeStruct((2 * N, DH), jnp.float32),
      mesh=_mesh(),
      scratch_types=_sc_scratch(DH),
      compiler_params=pltpu.CompilerParams(use_tc_tiling_on_sc=False),
  )(h, srcm, dstm)


def _sc_agg1_body(x16, srcm, dstm, out, *scratch):
  c = lax.axis_index("c")
  s = lax.axis_index("s")
  base = (c * NS + s) * RPT1
  _agg_pipeline(x16, srcm, dstm, out, scratch, 16, base, base, NIB1, s, c)


def _sc_agg1(x16, srcm, dstm):
  return pl.kernel(
      _sc_agg1_body,
      out_type=jax.ShapeDtypeStruct((2 * N, 16), jnp.float32),
      mesh=_mesh(),
      scratch_types=_sc_scratch(16),
      compiler_params=pltpu.CompilerParams(use_tc_tiling_on_sc=False),
  )(x16, srcm, dstm)


R = 5000
GRID = N // R
NB = N // R


def _full(shape):
  return pl.BlockSpec(shape, lambda *_: tuple(0 for _ in shape))


def _mlp_y(h0, h1, a0, a1, w1, b1, w2, b2, first):
  if first:
    hh = h0[:, :1] + a0[:, :1] + a1[:, :1]
    t = jnp.maximum(hh * w1[...] + b1[...], 0.0)
  else:
    hh = jnp.concatenate([h0[...] + a0[...], h1[...] + a1[...]], axis=1)
    t = jnp.maximum(
        jnp.dot(hh, w1[...], preferred_element_type=jnp.float32)
        + b1[...], 0.0)
  return jnp.dot(t, w2[...], preferred_element_type=jnp.float32) + b2[...]


def _bn_h(y, st_ref, g_ref, b_ref):
  mu = st_ref[0, :] / N
  var = st_ref[1, :] / N - mu * mu
  sc = g_ref[...] * lax.rsqrt(var + 1e-5)
  sh = b_ref[...] - mu * sc
  return jnp.maximum(y * sc + sh, 0.0)


def _layer_body(h0, h1, a0, a1, w1, b1, w2, b2, g, bb,
                out_ref, y_scr, st_scr, first):
  i = pl.program_id(0)
  r = i % GRID

  @pl.when(i == 0)
  def _():
    st_scr[...] = jnp.zeros((8, D), jnp.float32)

  @pl.when(i < GRID)
  def _():
    y = _mlp_y(h0, h1, a0, a1, w1, b1, w2, b2, first)
    y_scr[pl.ds(r * R, R), :] = y
    st_scr[...] += jnp.concatenate(
        [jnp.sum(y, axis=0)[None], jnp.sum(y * y, axis=0)[None],
         jnp.zeros((6, D), jnp.float32)], axis=0)

  @pl.when(i >= GRID)
  def _():
    h = _bn_h(y_scr[pl.ds(r * R, R), :], st_scr, g, bb)
    out_ref[0] = h[:, :DH]
    out_ref[1] = h[:, DH:]


def _layer4_body(h0, h1, a0, a1, w1, b1, w2, b2, g, bb, oh,
                 fw1, fb1, fw2, fb2, out_ref, y_scr, st_scr, ps_scr, pc_scr):
  i = pl.program_id(0)
  r = i % GRID

  @pl.when(i == 0)
  def _():
    st_scr[...] = jnp.zeros((8, D), jnp.float32)
    ps_scr[...] = jnp.zeros((G, D), jnp.float32)
    pc_scr[...] = jnp.zeros((8, G), jnp.float32)

  @pl.when(i < GRID)
  def _():
    y = _mlp_y(h0, h1, a0, a1, w1, b1, w2, b2, False)
    y_scr[pl.ds(r * R, R), :] = y
    st_scr[...] += jnp.concatenate(
        [jnp.sum(y, axis=0)[None], jnp.sum(y * y, axis=0)[None],
         jnp.zeros((6, D), jnp.float32)], axis=0)

  @pl.when(jnp.logical_and(i >= GRID, i < 2 * GRID))
  def _():
    h = _bn_h(y_scr[pl.ds(r * R, R), :], st_scr, g, bb)
    ohb = oh[0]
    ps_scr[...] += lax.dot_general(ohb, h, (((1,), (0,)), ((), ())),
                                   preferred_element_type=jnp.float32)
    pc_scr[...] += jnp.concatenate(
        [jnp.sum(ohb, axis=1)[None], jnp.zeros((7, G), jnp.float32)], axis=0)

  @pl.when(i == 2 * GRID)
  def _():
    cnt = jnp.maximum(pc_scr[0, :], 1.0)[:, None]
    pooled = ps_scr[...] / cnt
    rr = jnp.maximum(
        jnp.dot(pooled, fw1[...], preferred_element_type=jnp.float32)
        + fb1[...], 0.0)
    out_ref[...] = jnp.dot(rr, fw2[...],
                           preferred_element_type=jnp.float32) + fb2[...]


def _row_spec(w, second_half):
  off = NB if second_half else 0

  def imap(i):
    return (jnp.where(i < GRID, i % GRID + off, off), 0)
  return pl.BlockSpec((R, w), imap)


def _layer(hf, aggf, w1, b1, w2, b2, g, bb, first):
  wh = 16 if first else DH
  body = functools.partial(_layer_body, first=first)
  return pl.pallas_call(
      body,
      grid=(2 * GRID,),
      in_specs=[
          _row_spec(16 if first else DH, False),
          _row_spec(16 if first else DH, not first),
          _row_spec(wh, False),
          _row_spec(wh, True),
          _full((1, D) if first else (D, D)), _full((1, D)),
          _full((D, D)), _full((1, D)),
          _full((1, D)), _full((1, D)),
      ],
      out_specs=pl.BlockSpec(
          (2, R, DH), lambda i: (0, jnp.where(i >= GRID, i % GRID, 0), 0)),
      out_shape=jax.ShapeDtypeStruct((2, N, DH), jnp.float32),
      scratch_shapes=[
          pltpu.VMEM((N, D), jnp.float32),
          pltpu.VMEM((8, D), jnp.float32),
      ],
  )(hf, hf, aggf, aggf, w1, b1, w2, b2, g, bb)


def _layer4(hf, aggf, w1, b1, w2, b2, g, bb, oh, fw1, fb1, fw2, fb2):
  def oh_map(i):
    return (jnp.where(jnp.logical_and(i >= GRID, i < 2 * GRID),
                      i % GRID, 0), 0, 0)
  return pl.pallas_call(
      _layer4_body,
      grid=(2 * GRID + 1,),
      in_specs=[
          _row_spec(DH, False), _row_spec(DH, True),
          _row_spec(DH, False), _row_spec(DH, True),
          _full((D, D)), _full((1, D)), _full((D, D)), _full((1, D)),
          _full((1, D)), _full((1, D)),
          pl.BlockSpec((1, G, R), oh_map),
          _full((D, 128)), _full((1, 128)), _full((128, 10)), _full((1, 10)),
      ],
      out_specs=_full((G, 10)),
      out_shape=jax.ShapeDtypeStruct((G, 10), jnp.float32),
      scratch_shapes=[
          pltpu.VMEM((N, D), jnp.float32),
          pltpu.VMEM((8, D), jnp.float32),
          pltpu.VMEM((G, D), jnp.float32),
          pltpu.VMEM((8, G), jnp.float32),
      ],
  )(hf, hf, aggf, aggf, w1, b1, w2, b2, g, bb, oh, fw1, fb1, fw2, fb2)


def kernel(x, edge_index, batch,
           w1_1, b1_1, w1_2, b1_2, bn1_g, bn1_b,
           w2_1, b2_1, w2_2, b2_2, bn2_g, bn2_b,
           w3_1, b3_1, w3_2, b3_2, bn3_g, bn3_b,
           w4_1, b4_1, w4_2, b4_2, bn4_g, bn4_b,
           fc1_w, fc1_b, fc2_w, fc2_b):
  src = edge_index[0]
  srcm = jnp.concatenate([src, src + N]).reshape(2 * NROWS, BE)
  dstm = edge_index[1].reshape(NROWS, BE)

  x16 = jnp.pad(x, ((0, 0), (0, 15)))
  agg1 = _sc_agg1(x16, srcm, dstm)
  h = _layer(x16, agg1, w1_1, b1_1.reshape(1, D), w1_2, b1_2.reshape(1, D),
             bn1_g.reshape(1, D), bn1_b.reshape(1, D),
             first=True).reshape(2 * N, DH)

  for (w1, b1, w2, b2, g, b) in (
      (w2_1, b2_1, w2_2, b2_2, bn2_g, bn2_b),
      (w3_1, b3_1, w3_2, b3_2, bn3_g, bn3_b)):
    aggf = _sc_agg(h, srcm, dstm)
    h = _layer(h, aggf, w1, b1.reshape(1, D), w2, b2.reshape(1, D),
               g.reshape(1, D), b.reshape(1, D),
               first=False).reshape(2 * N, DH)

  aggf = _sc_agg(h, srcm, dstm)
  oh = (batch.reshape(GRID, 1, R)
        == jnp.arange(G, dtype=jnp.int32)[None, :, None]
        ).astype(jnp.float32)
  return _layer4(h, aggf, w4_1, b4_1.reshape(1, D), w4_2, b4_2.reshape(1, D),
                 bn4_g.reshape(1, D), bn4_b.reshape(1, D), oh,
                 fc1_w, fc1_b.reshape(1, 128), fc2_w, fc2_b.reshape(1, 10))


# speedup vs baseline: 1.0075x; 1.0075x over previous
import functools

import jax
import jax.numpy as jnp
from jax import lax
from jax.experimental import pallas as pl
from jax.experimental.pallas import tpu as pltpu
from jax.experimental.pallas import tpu_sc as plsc

N = 50000
E = 800000
G = 64
D = 64
DH = 32
NC = 2
NS = 16
BE = 125
NROWS = E // BE
RPT = NROWS // NS
RPT1 = NROWS // (NC * NS)
NPT = N // NS

_mesh_cache = []


def _mesh():
  if not _mesh_cache:
    _mesh_cache.append(plsc.VectorSubcoreMesh(
        core_axis_name="c", subcore_axis_name="s",
        num_cores=NC, num_subcores=NS))
  return _mesh_cache[0]


IDXB = 20
RING = 4
NIB = RPT // IDXB
NIB1 = RPT1 // IDXB


def _agg_pipeline(tbl, srcm, dstm, out, scratch, dw, base, dbase, nib, s, c):
  (is0, is1, id0, id1, r0, r1, r2, r3, acc,
   gs0, gs1, gs2, gs3, isem) = scratch
  iss = (is0, is1)
  ids = (id0, id1)
  ring = (r0, r1, r2, r3)
  gsem = (gs0, gs1, gs2, gs3)

  def zb(i, carry):
    ring[0][i, pl.ds(0, 16)] = jnp.zeros((16,), jnp.float32)
    if dw > 16:
      ring[0][i, pl.ds(16, 16)] = jnp.zeros((16,), jnp.float32)
    return carry
  lax.fori_loop(0, BE, zb, 0)

  def zc(k, carry):
    pltpu.sync_copy(ring[0], acc.at[pl.ds(s * NPT + k * BE, BE)])
    return carry
  lax.fori_loop(0, NPT // BE, zc, 0)
  plsc.subcore_barrier()

  def idx_load(nb, b, sync):
    sc_ = pltpu.async_copy(srcm.at[pl.ds(base + nb * IDXB, IDXB)],
                           iss[b], isem)
    dc_ = pltpu.async_copy(dstm.at[pl.ds(dbase + nb * IDXB, IDXB)],
                           ids[b], isem)
    if sync:
      sc_.wait()
      dc_.wait()

  def idx_wait(nb, b):
    pltpu.make_async_copy(srcm.at[pl.ds(0, IDXB)], iss[b], isem).wait()
    pltpu.make_async_copy(dstm.at[pl.ds(0, IDXB)], ids[b], isem).wait()

  def fire(b, q, r):
    pltpu.async_copy(tbl.at[iss[b].at[q]], ring[r], gsem[r])

  def drain(r):
    pltpu.make_async_copy(tbl.at[pl.ds(0, BE)], ring[r], gsem[r]).wait()

  idx_load(0, 0, True)

  def outer(nb0, carry):
    for b in range(2):
      nb = nb0 * 2 + b

      @pl.when(nb + 1 < nib)
      def _():
        idx_load(nb + 1, 1 - b, False)
      for q in range(RING):
        fire(b, q, q)
      for q in range(IDXB):
        r = q % RING
        drain(r)
        pltpu.sync_copy(ring[r], acc.at[ids[b].at[q]], add=True)
        if q + RING < IDXB:
          fire(b, q + RING, r)

      @pl.when(nb + 1 < nib)
      def _():
        idx_wait(nb + 1, 1 - b)
    return carry
  lax.fori_loop(0, nib // 2, outer, 0)
  plsc.subcore_barrier()

  pltpu.sync_copy(acc.at[pl.ds(s * NPT, NPT)],
                  out.at[pl.ds(c * N + s * NPT, NPT)])


def _sc_agg_body(tbl, srcm, dstm, out, *scratch):
  c = lax.axis_index("c")
  s = lax.axis_index("s")
  _agg_pipeline(tbl, srcm, dstm, out, scratch, DH,
                c * NROWS + s * RPT, s * RPT, NIB, s, c)


def _sc_scratch(dw):
  return ([pltpu.VMEM((IDXB, BE), jnp.int32)] * 4
          + [pltpu.VMEM((BE, dw), jnp.float32)] * RING
          + [pltpu.VMEM_SHARED((N, dw), jnp.float32)]
          + [pltpu.SemaphoreType.DMA] * (RING + 1))


def _sc_agg(h, srcm, dstm):
  return pl.kernel(
      _sc_agg_body,
      out_type=jax.ShapeDtypeStruct((2 * N, DH), jnp.float32),
      mesh=_mesh(),
      scratch_types=_sc_scratch(DH),
      compiler_params=pltpu.CompilerParams(use_tc_tiling_on_sc=False),
  )(h, srcm, dstm)


def _sc_agg1_body(x16, srcm, dstm, out, *scratch):
  c = lax.axis_index("c")
  s = lax.axis_index("s")
  base = (c * NS + s) * RPT1
  _agg_pipeline(x16, srcm, dstm, out, scratch, 16, base, base, NIB1, s, c)


def _sc_agg1(x16, srcm, dstm):
  return pl.kernel(
      _sc_agg1_body,
      out_type=jax.ShapeDtypeStruct((2 * N, 16), jnp.float32),
      mesh=_mesh(),
      scratch_types=_sc_scratch(16),
      compiler_params=pltpu.CompilerParams(use_tc_tiling_on_sc=False),
  )(x16, srcm, dstm)


R = 5000
GRID = N // R
NB = N // R


def _full(shape):
  return pl.BlockSpec(shape, lambda *_: tuple(0 for _ in shape))


def _mlp_y(h0, h1, a0, a1, w1, b1, w2, b2, first):
  if first:
    hh = h0[:, :1] + a0[:, :1] + a1[:, :1]
    t = jnp.maximum(hh * w1[...] + b1[...], 0.0)
  else:
    hh = jnp.concatenate([h0[...] + a0[...], h1[...] + a1[...]], axis=1)
    t = jnp.maximum(
        jnp.dot(hh, w1[...], preferred_element_type=jnp.float32)
        + b1[...], 0.0)
  return jnp.dot(t, w2[...], preferred_element_type=jnp.float32) + b2[...]


def _bn_h(y, st_ref, g_ref, b_ref):
  mu = st_ref[0, :] / N
  var = st_ref[1, :] / N - mu * mu
  sc = g_ref[...] * lax.rsqrt(var + 1e-5)
  sh = b_ref[...] - mu * sc
  return jnp.maximum(y * sc + sh, 0.0)


def _layer_body(h0, h1, a0, a1, w1, b1, w2, b2, g, bb,
                out_ref, y_scr, st_scr, first):
  i = pl.program_id(0)
  r = i % GRID

  @pl.when(i == 0)
  def _():
    st_scr[...] = jnp.zeros((8, D), jnp.float32)

  @pl.when(i < GRID)
  def _():
    y = _mlp_y(h0, h1, a0, a1, w1, b1, w2, b2, first)
    y_scr[pl.ds(r * R, R), :] = y
    st_scr[...] += jnp.concatenate(
        [jnp.sum(y, axis=0)[None], jnp.sum(y * y, axis=0)[None],
         jnp.zeros((6, D), jnp.float32)], axis=0)

  @pl.when(i >= GRID)
  def _():
    h = _bn_h(y_scr[pl.ds(r * R, R), :], st_scr, g, bb)
    out_ref[0] = h[:, :DH]
    out_ref[1] = h[:, DH:]


def _layer4_body(h0, h1, a0, a1, w1, b1, w2, b2, g, bb, oh,
                 fw1, fb1, fw2, fb2, out_ref, y_scr, st_scr, ps_scr, pc_scr):
  i = pl.program_id(0)
  r = i % GRID

  @pl.when(i == 0)
  def _():
    st_scr[...] = jnp.zeros((8, D), jnp.float32)
    ps_scr[...] = jnp.zeros((G, D), jnp.float32)
    pc_scr[...] = jnp.zeros((8, G), jnp.float32)

  @pl.when(i < GRID)
  def _():
    y = _mlp_y(h0, h1, a0, a1, w1, b1, w2, b2, False)
    y_scr[pl.ds(r * R, R), :] = y
    st_scr[...] += jnp.concatenate(
        [jnp.sum(y, axis=0)[None], jnp.sum(y * y, axis=0)[None],
         jnp.zeros((6, D), jnp.float32)], axis=0)

  @pl.when(jnp.logical_and(i >= GRID, i < 2 * GRID))
  def _():
    h = _bn_h(y_scr[pl.ds(r * R, R), :], st_scr, g, bb)
    ohb = oh[0]
    ps_scr[...] += lax.dot_general(ohb, h, (((1,), (0,)), ((), ())),
                                   preferred_element_type=jnp.float32)
    pc_scr[...] += jnp.concatenate(
        [jnp.sum(ohb, axis=1)[None], jnp.zeros((7, G), jnp.float32)], axis=0)

  @pl.when(i == 2 * GRID)
  def _():
    cnt = jnp.maximum(pc_scr[0, :], 1.0)[:, None]
    pooled = ps_scr[...] / cnt
    rr = jnp.maximum(
        jnp.dot(pooled, fw1[...], preferred_element_type=jnp.float32)
        + fb1[...], 0.0)
    out_ref[...] = jnp.dot(rr, fw2[...],
                           preferred_element_type=jnp.float32) + fb2[...]


def _row_spec(w, second_half):
  off = NB if second_half else 0

  def imap(i):
    return (jnp.where(i < GRID, i % GRID + off, off), 0)
  return pl.BlockSpec((R, w), imap)


def _layer(hf, aggf, w1, b1, w2, b2, g, bb, first):
  wh = 16 if first else DH
  body = functools.partial(_layer_body, first=first)
  return pl.pallas_call(
      body,
      grid=(2 * GRID,),
      in_specs=[
          _row_spec(16 if first else DH, False),
          _row_spec(16 if first else DH, not first),
          _row_spec(wh, False),
          _row_spec(wh, True),
          _full((1, D) if first else (D, D)), _full((1, D)),
          _full((D, D)), _full((1, D)),
          _full((1, D)), _full((1, D)),
      ],
      out_specs=pl.BlockSpec(
          (2, R, DH), lambda i: (0, jnp.where(i >= GRID, i % GRID, 0), 0)),
      out_shape=jax.ShapeDtypeStruct((2, N, DH), jnp.float32),
      scratch_shapes=[
          pltpu.VMEM((N, D), jnp.float32),
          pltpu.VMEM((8, D), jnp.float32),
      ],
  )(hf, hf, aggf, aggf, w1, b1, w2, b2, g, bb)


def _layer4(hf, aggf, w1, b1, w2, b2, g, bb, oh, fw1, fb1, fw2, fb2):
  def oh_map(i):
    return (jnp.where(jnp.logical_and(i >= GRID, i < 2 * GRID),
                      i % GRID, 0), 0, 0)
  return pl.pallas_call(
      _layer4_body,
      grid=(2 * GRID + 1,),
      in_specs=[
          _row_spec(DH, False), _row_spec(DH, True),
          _row_spec(DH, False), _row_spec(DH, True),
          _full((D, D)), _full((1, D)), _full((D, D)), _full((1, D)),
          _full((1, D)), _full((1, D)),
          pl.BlockSpec((1, G, R), oh_map),
          _full((D, 128)), _full((1, 128)), _full((128, 10)), _full((1, 10)),
      ],
      out_specs=_full((G, 10)),
      out_shape=jax.ShapeDtypeStruct((G, 10), jnp.float32),
      scratch_shapes=[
          pltpu.VMEM((N, D), jnp.float32),
          pltpu.VMEM((8, D), jnp.float32),
          pltpu.VMEM((G, D), jnp.float32),
          pltpu.VMEM((8, G), jnp.float32),
      ],
  )(hf, hf, aggf, aggf, w1, b1, w2, b2, g, bb, oh, fw1, fb1, fw2, fb2)


def kernel(x, edge_index, batch,
           w1_1, b1_1, w1_2, b1_2, bn1_g, bn1_b,
           w2_1, b2_1, w2_2, b2_2, bn2_g, bn2_b,
           w3_1, b3_1, w3_2, b3_2, bn3_g, bn3_b,
           w4_1, b4_1, w4_2, b4_2, bn4_g, bn4_b,
           fc1_w, fc1_b, fc2_w, fc2_b):
  src = edge_index[0]
  srcm = jnp.concatenate([src, src + N]).reshape(2 * NROWS, BE)
  dstm = edge_index[1].reshape(NROWS, BE)

  x16 = jnp.pad(x, ((0, 0), (0, 15)))
  agg1 = _sc_agg1(x16, srcm, dstm)
  h = _layer(x16, agg1, w1_1, b1_1.reshape(1, D), w1_2, b1_2.reshape(1, D),
             bn1_g.reshape(1, D), bn1_b.reshape(1, D),
             first=True).reshape(2 * N, DH)

  for (w1, b1, w2, b2, g, b) in (
      (w2_1, b2_1, w2_2, b2_2, bn2_g, bn2_b),
      (w3_1, b3_1, w3_2, b3_2, bn3_g, bn3_b)):
    aggf = _sc_agg(h, srcm, dstm)
    h = _layer(h, aggf, w1, b1.reshape(1, D), w2, b2.reshape(1, D),
               g.reshape(1, D), b.reshape(1, D),
               first=False).reshape(2 * N, DH)

  aggf = _sc_agg(h, srcm, dstm)
  oh = (batch.reshape(GRID, 1, R)
        == jnp.arange(G, dtype=jnp.int32)[None, :, None]
        ).astype(jnp.float32)
  return _layer4(h, aggf, w4_1, b4_1.reshape(1, D), w4_2, b4_2.reshape(1, D),
                 bn4_g.reshape(1, D), bn4_b.reshape(1, D), oh,
                 fc1_w, fc1_b.reshape(1, 128), fc2_w, fc2_b.reshape(1, 10))


# speedup vs baseline: 1.0447x; 1.0370x over previous
import functools

import jax
import jax.numpy as jnp
from jax import lax
from jax.experimental import pallas as pl
from jax.experimental.pallas import tpu as pltpu
from jax.experimental.pallas import tpu_sc as plsc

N = 50000
E = 800000
G = 64
D = 64
DH = 32
NC = 2
NS = 16
BE = 125
NROWS = E // BE
RPT = NROWS // NS
RPT1 = NROWS // (NC * NS)
NPT = N // NS

_mesh_cache = []


def _mesh():
  if not _mesh_cache:
    _mesh_cache.append(plsc.VectorSubcoreMesh(
        core_axis_name="c", subcore_axis_name="s",
        num_cores=NC, num_subcores=NS))
  return _mesh_cache[0]


IDXB = 20
RING = 5
NIB = RPT // IDXB
NIB1 = RPT1 // IDXB


def _agg_pipeline(tbl, srcm, dstm, out, scratch, dw, base, dbase, nib, s, c):
  (is0, is1, id0, id1, r0, r1, r2, r3, r4, acc,
   gs0, gs1, gs2, gs3, gs4, isem) = scratch
  iss = (is0, is1)
  ids = (id0, id1)
  ring = (r0, r1, r2, r3, r4)
  gsem = (gs0, gs1, gs2, gs3, gs4)

  def zb(i, carry):
    ring[0][i, pl.ds(0, 16)] = jnp.zeros((16,), jnp.float32)
    if dw > 16:
      ring[0][i, pl.ds(16, 16)] = jnp.zeros((16,), jnp.float32)
    return carry
  lax.fori_loop(0, BE, zb, 0)

  def zc(k, carry):
    pltpu.sync_copy(ring[0], acc.at[pl.ds(s * NPT + k * BE, BE)])
    return carry
  lax.fori_loop(0, NPT // BE, zc, 0)
  plsc.subcore_barrier()

  def idx_load(nb, b, sync):
    sc_ = pltpu.async_copy(srcm.at[pl.ds(base + nb * IDXB, IDXB)],
                           iss[b], isem)
    dc_ = pltpu.async_copy(dstm.at[pl.ds(dbase + nb * IDXB, IDXB)],
                           ids[b], isem)
    if sync:
      sc_.wait()
      dc_.wait()

  def idx_wait(nb, b):
    pltpu.make_async_copy(srcm.at[pl.ds(0, IDXB)], iss[b], isem).wait()
    pltpu.make_async_copy(dstm.at[pl.ds(0, IDXB)], ids[b], isem).wait()

  def fire(b, q, r):
    pltpu.async_copy(tbl.at[iss[b].at[q]], ring[r], gsem[r])

  def drain(r):
    pltpu.make_async_copy(tbl.at[pl.ds(0, BE)], ring[r], gsem[r]).wait()

  idx_load(0, 0, True)

  def outer(nb0, carry):
    for b in range(2):
      nb = nb0 * 2 + b

      @pl.when(nb + 1 < nib)
      def _():
        idx_load(nb + 1, 1 - b, False)
      for q in range(RING):
        fire(b, q, q)
      for q in range(IDXB):
        r = q % RING
        drain(r)
        pltpu.sync_copy(ring[r], acc.at[ids[b].at[q]], add=True)
        if q + RING < IDXB:
          fire(b, q + RING, r)

      @pl.when(nb + 1 < nib)
      def _():
        idx_wait(nb + 1, 1 - b)
    return carry
  lax.fori_loop(0, nib // 2, outer, 0)
  plsc.subcore_barrier()

  pltpu.sync_copy(acc.at[pl.ds(s * NPT, NPT)],
                  out.at[pl.ds(c * N + s * NPT, NPT)])


def _sc_agg_body(tbl, srcm, dstm, out, *scratch):
  c = lax.axis_index("c")
  s = lax.axis_index("s")
  _agg_pipeline(tbl, srcm, dstm, out, scratch, DH,
                c * NROWS + s * RPT, s * RPT, NIB, s, c)


def _sc_scratch(dw):
  return ([pltpu.VMEM((IDXB, BE), jnp.int32)] * 4
          + [pltpu.VMEM((BE, dw), jnp.float32)] * RING
          + [pltpu.VMEM_SHARED((N, dw), jnp.float32)]
          + [pltpu.SemaphoreType.DMA] * (RING + 1))


def _sc_agg(h, srcm, dstm):
  return pl.kernel(
      _sc_agg_body,
      out_type=jax.ShapeDtypeStruct((2 * N, DH), jnp.float32),
      mesh=_mesh(),
      scratch_types=_sc_scratch(DH),
      compiler_params=pltpu.CompilerParams(use_tc_tiling_on_sc=False),
  )(h, srcm, dstm)


def _sc_agg1_body(x16, srcm, dstm, out, *scratch):
  c = lax.axis_index("c")
  s = lax.axis_index("s")
  base = (c * NS + s) * RPT1
  _agg_pipeline(x16, srcm, dstm, out, scratch, 16, base, base, NIB1, s, c)


def _sc_agg1(x16, srcm, dstm):
  return pl.kernel(
      _sc_agg1_body,
      out_type=jax.ShapeDtypeStruct((2 * N, 16), jnp.float32),
      mesh=_mesh(),
      scratch_types=_sc_scratch(16),
      compiler_params=pltpu.CompilerParams(use_tc_tiling_on_sc=False),
  )(x16, srcm, dstm)


R = 5000
GRID = N // R
NB = N // R


def _full(shape):
  return pl.BlockSpec(shape, lambda *_: tuple(0 for _ in shape))


def _mlp_y(h0, h1, a0, a1, w1, b1, w2, b2, first):
  if first:
    hh = h0[:, :1] + a0[:, :1] + a1[:, :1]
    t = jnp.maximum(hh * w1[...] + b1[...], 0.0)
  else:
    hh = jnp.concatenate([h0[...] + a0[...], h1[...] + a1[...]], axis=1)
    t = jnp.maximum(
        jnp.dot(hh, w1[...], preferred_element_type=jnp.float32)
        + b1[...], 0.0)
  return jnp.dot(t, w2[...], preferred_element_type=jnp.float32) + b2[...]


def _bn_h(y, st_ref, g_ref, b_ref):
  mu = st_ref[0, :] / N
  var = st_ref[1, :] / N - mu * mu
  sc = g_ref[...] * lax.rsqrt(var + 1e-5)
  sh = b_ref[...] - mu * sc
  return jnp.maximum(y * sc + sh, 0.0)


def _layer_body(h0, h1, a0, a1, w1, b1, w2, b2, g, bb,
                out_ref, y_scr, st_scr, first):
  i = pl.program_id(0)
  r = i % GRID

  @pl.when(i == 0)
  def _():
    st_scr[...] = jnp.zeros((8, D), jnp.float32)

  @pl.when(i < GRID)
  def _():
    y = _mlp_y(h0, h1, a0, a1, w1, b1, w2, b2, first)
    y_scr[pl.ds(r * R, R), :] = y
    st_scr[...] += jnp.concatenate(
        [jnp.sum(y, axis=0)[None], jnp.sum(y * y, axis=0)[None],
         jnp.zeros((6, D), jnp.float32)], axis=0)

  @pl.when(i >= GRID)
  def _():
    h = _bn_h(y_scr[pl.ds(r * R, R), :], st_scr, g, bb)
    out_ref[0] = h[:, :DH]
    out_ref[1] = h[:, DH:]


def _layer4_body(h0, h1, a0, a1, w1, b1, w2, b2, g, bb, oh,
                 fw1, fb1, fw2, fb2, out_ref, y_scr, st_scr, ps_scr, pc_scr):
  i = pl.program_id(0)
  r = i % GRID

  @pl.when(i == 0)
  def _():
    st_scr[...] = jnp.zeros((8, D), jnp.float32)
    ps_scr[...] = jnp.zeros((G, D), jnp.float32)
    pc_scr[...] = jnp.zeros((8, G), jnp.float32)

  @pl.when(i < GRID)
  def _():
    y = _mlp_y(h0, h1, a0, a1, w1, b1, w2, b2, False)
    y_scr[pl.ds(r * R, R), :] = y
    st_scr[...] += jnp.concatenate(
        [jnp.sum(y, axis=0)[None], jnp.sum(y * y, axis=0)[None],
         jnp.zeros((6, D), jnp.float32)], axis=0)

  @pl.when(jnp.logical_and(i >= GRID, i < 2 * GRID))
  def _():
    h = _bn_h(y_scr[pl.ds(r * R, R), :], st_scr, g, bb)
    ohb = oh[0]
    ps_scr[...] += lax.dot_general(ohb, h, (((1,), (0,)), ((), ())),
                                   preferred_element_type=jnp.float32)
    pc_scr[...] += jnp.concatenate(
        [jnp.sum(ohb, axis=1)[None], jnp.zeros((7, G), jnp.float32)], axis=0)

  @pl.when(i == 2 * GRID)
  def _():
    cnt = jnp.maximum(pc_scr[0, :], 1.0)[:, None]
    pooled = ps_scr[...] / cnt
    rr = jnp.maximum(
        jnp.dot(pooled, fw1[...], preferred_element_type=jnp.float32)
        + fb1[...], 0.0)
    out_ref[...] = jnp.dot(rr, fw2[...],
                           preferred_element_type=jnp.float32) + fb2[...]


def _row_spec(w, second_half):
  off = NB if second_half else 0

  def imap(i):
    return (jnp.where(i < GRID, i % GRID + off, off), 0)
  return pl.BlockSpec((R, w), imap)


def _layer(hf, aggf, w1, b1, w2, b2, g, bb, first):
  wh = 16 if first else DH
  body = functools.partial(_layer_body, first=first)
  return pl.pallas_call(
      body,
      grid=(2 * GRID,),
      in_specs=[
          _row_spec(16 if first else DH, False),
          _row_spec(16 if first else DH, not first),
          _row_spec(wh, False),
          _row_spec(wh, True),
          _full((1, D) if first else (D, D)), _full((1, D)),
          _full((D, D)), _full((1, D)),
          _full((1, D)), _full((1, D)),
      ],
      out_specs=pl.BlockSpec(
          (2, R, DH), lambda i: (0, jnp.where(i >= GRID, i % GRID, 0), 0)),
      out_shape=jax.ShapeDtypeStruct((2, N, DH), jnp.float32),
      scratch_shapes=[
          pltpu.VMEM((N, D), jnp.float32),
          pltpu.VMEM((8, D), jnp.float32),
      ],
  )(hf, hf, aggf, aggf, w1, b1, w2, b2, g, bb)


def _layer4(hf, aggf, w1, b1, w2, b2, g, bb, oh, fw1, fb1, fw2, fb2):
  def oh_map(i):
    return (jnp.where(jnp.logical_and(i >= GRID, i < 2 * GRID),
                      i % GRID, 0), 0, 0)
  return pl.pallas_call(
      _layer4_body,
      grid=(2 * GRID + 1,),
      in_specs=[
          _row_spec(DH, False), _row_spec(DH, True),
          _row_spec(DH, False), _row_spec(DH, True),
          _full((D, D)), _full((1, D)), _full((D, D)), _full((1, D)),
          _full((1, D)), _full((1, D)),
          pl.BlockSpec((1, G, R), oh_map),
          _full((D, 128)), _full((1, 128)), _full((128, 10)), _full((1, 10)),
      ],
      out_specs=_full((G, 10)),
      out_shape=jax.ShapeDtypeStruct((G, 10), jnp.float32),
      scratch_shapes=[
          pltpu.VMEM((N, D), jnp.float32),
          pltpu.VMEM((8, D), jnp.float32),
          pltpu.VMEM((G, D), jnp.float32),
          pltpu.VMEM((8, G), jnp.float32),
      ],
  )(hf, hf, aggf, aggf, w1, b1, w2, b2, g, bb, oh, fw1, fb1, fw2, fb2)


def kernel(x, edge_index, batch,
           w1_1, b1_1, w1_2, b1_2, bn1_g, bn1_b,
           w2_1, b2_1, w2_2, b2_2, bn2_g, bn2_b,
           w3_1, b3_1, w3_2, b3_2, bn3_g, bn3_b,
           w4_1, b4_1, w4_2, b4_2, bn4_g, bn4_b,
           fc1_w, fc1_b, fc2_w, fc2_b):
  src = edge_index[0]
  srcm = jnp.concatenate([src, src + N]).reshape(2 * NROWS, BE)
  dstm = edge_index[1].reshape(NROWS, BE)

  x16 = jnp.pad(x, ((0, 0), (0, 15)))
  agg1 = _sc_agg1(x16, srcm, dstm)
  h = _layer(x16, agg1, w1_1, b1_1.reshape(1, D), w1_2, b1_2.reshape(1, D),
             bn1_g.reshape(1, D), bn1_b.reshape(1, D),
             first=True).reshape(2 * N, DH)

  for (w1, b1, w2, b2, g, b) in (
      (w2_1, b2_1, w2_2, b2_2, bn2_g, bn2_b),
      (w3_1, b3_1, w3_2, b3_2, bn3_g, bn3_b)):
    aggf = _sc_agg(h, srcm, dstm)
    h = _layer(h, aggf, w1, b1.reshape(1, D), w2, b2.reshape(1, D),
               g.reshape(1, D), b.reshape(1, D),
               first=False).reshape(2 * N, DH)

  aggf = _sc_agg(h, srcm, dstm)
  oh = (batch.reshape(GRID, 1, R)
        == jnp.arange(G, dtype=jnp.int32)[None, :, None]
        ).astype(jnp.float32)
  return _layer4(h, aggf, w4_1, b4_1.reshape(1, D), w4_2, b4_2.reshape(1, D),
                 bn4_g.reshape(1, D), bn4_b.reshape(1, D), oh,
                 fc1_w, fc1_b.reshape(1, 128), fc2_w, fc2_b.reshape(1, 10))


# speedup vs baseline: 1.0552x; 1.0101x over previous
import functools

import jax
import jax.numpy as jnp
from jax import lax
from jax.experimental import pallas as pl
from jax.experimental.pallas import tpu as pltpu
from jax.experimental.pallas import tpu_sc as plsc

N = 50000
E = 800000
G = 64
D = 64
DH = 32
NC = 2
NS = 16
BE = 125
NROWS = E // BE
RPT = NROWS // NS
RPT1 = NROWS // (NC * NS)
NPT = N // NS

_mesh_cache = []


def _mesh():
  if not _mesh_cache:
    _mesh_cache.append(plsc.VectorSubcoreMesh(
        core_axis_name="c", subcore_axis_name="s",
        num_cores=NC, num_subcores=NS))
  return _mesh_cache[0]


IDXB = 20
RING = 5
NIB = RPT // IDXB
NIB1 = RPT1 // IDXB


def _agg_pipeline(tbl, srcm, dstm, out, scratch, dw, base, dbase, nib, s, c):
  (is0, is1, id0, id1, r0, r1, r2, r3, r4, acc,
   gs0, gs1, gs2, gs3, gs4, isem) = scratch
  iss = (is0, is1)
  ids = (id0, id1)
  ring = (r0, r1, r2, r3, r4)
  gsem = (gs0, gs1, gs2, gs3, gs4)

  def zb(i, carry):
    ring[0][i, pl.ds(0, 16)] = jnp.zeros((16,), jnp.float32)
    if dw > 16:
      ring[0][i, pl.ds(16, 16)] = jnp.zeros((16,), jnp.float32)
    return carry
  lax.fori_loop(0, BE, zb, 0)

  def zc(k, carry):
    pltpu.sync_copy(ring[0], acc.at[pl.ds(s * NPT + k * BE, BE)])
    return carry
  lax.fori_loop(0, NPT // BE, zc, 0)
  plsc.subcore_barrier()

  def idx_load(nb, b, sync):
    sc_ = pltpu.async_copy(srcm.at[pl.ds(base + nb * IDXB, IDXB)],
                           iss[b], isem)
    dc_ = pltpu.async_copy(dstm.at[pl.ds(dbase + nb * IDXB, IDXB)],
                           ids[b], isem)
    if sync:
      sc_.wait()
      dc_.wait()

  def idx_wait(nb, b):
    pltpu.make_async_copy(srcm.at[pl.ds(0, IDXB)], iss[b], isem).wait()
    pltpu.make_async_copy(dstm.at[pl.ds(0, IDXB)], ids[b], isem).wait()

  def fire(b, q, r):
    pltpu.async_copy(tbl.at[iss[b].at[q]], ring[r], gsem[r])

  def drain(r):
    pltpu.make_async_copy(tbl.at[pl.ds(0, BE)], ring[r], gsem[r]).wait()

  idx_load(0, 0, True)

  def outer(nb0, carry):
    for b in range(2):
      nb = nb0 * 2 + b

      @pl.when(nb + 1 < nib)
      def _():
        idx_load(nb + 1, 1 - b, False)
      for q in range(RING):
        fire(b, q, q)
      for q in range(IDXB):
        r = q % RING
        drain(r)
        pltpu.sync_copy(ring[r], acc.at[ids[b].at[q]], add=True)
        if q + RING < IDXB:
          fire(b, q + RING, r)

      @pl.when(nb + 1 < nib)
      def _():
        idx_wait(nb + 1, 1 - b)
    return carry
  lax.fori_loop(0, nib // 2, outer, 0)
  plsc.subcore_barrier()

  pltpu.sync_copy(acc.at[pl.ds(s * NPT, NPT)],
                  out.at[pl.ds(c * N + s * NPT, NPT)])


def _sc_agg_body(tbl, srcm, dstm, out, *scratch):
  c = lax.axis_index("c")
  s = lax.axis_index("s")
  _agg_pipeline(tbl, srcm, dstm, out, scratch, DH,
                c * NROWS + s * RPT, s * RPT, NIB, s, c)


def _sc_scratch(dw):
  return ([pltpu.VMEM((IDXB, BE), jnp.int32)] * 4
          + [pltpu.VMEM((BE, dw), jnp.float32)] * RING
          + [pltpu.VMEM_SHARED((N, dw), jnp.float32)]
          + [pltpu.SemaphoreType.DMA] * (RING + 1))


def _sc_agg(h, srcm, dstm):
  return pl.kernel(
      _sc_agg_body,
      out_type=jax.ShapeDtypeStruct((2 * N, DH), jnp.float32),
      mesh=_mesh(),
      scratch_types=_sc_scratch(DH),
      compiler_params=pltpu.CompilerParams(use_tc_tiling_on_sc=False),
  )(h, srcm, dstm)


def _sc_agg1_body(x16, srcm, dstm, out, *scratch):
  c = lax.axis_index("c")
  s = lax.axis_index("s")
  base = (c * NS + s) * RPT1
  _agg_pipeline(x16, srcm, dstm, out, scratch, 16, base, base, NIB1, s, c)


def _sc_agg1(x16, srcm, dstm):
  return pl.kernel(
      _sc_agg1_body,
      out_type=jax.ShapeDtypeStruct((2 * N, 16), jnp.float32),
      mesh=_mesh(),
      scratch_types=_sc_scratch(16),
      compiler_params=pltpu.CompilerParams(use_tc_tiling_on_sc=False),
  )(x16, srcm, dstm)


R = 5000
GRID = N // R
NB = N // R


def _full(shape):
  return pl.BlockSpec(shape, lambda *_: tuple(0 for _ in shape))


def _mlp_y(h3, a3, w1, b1, w2, b2, first):
  if first:
    hh = h3[:, :1] + a3[0, :, :1] + a3[1, :, :1]
    t = jnp.maximum(hh * w1[...] + b1[...], 0.0)
  else:
    hh = jnp.concatenate([h3[0] + a3[0], h3[1] + a3[1]], axis=1)
    t = jnp.maximum(
        jnp.dot(hh, w1[...], preferred_element_type=jnp.float32)
        + b1[...], 0.0)
  return jnp.dot(t, w2[...], preferred_element_type=jnp.float32) + b2[...]


def _bn_h(y, st_ref, g_ref, b_ref):
  mu = st_ref[0, :] / N
  var = st_ref[1, :] / N - mu * mu
  sc = g_ref[...] * lax.rsqrt(var + 1e-5)
  sh = b_ref[...] - mu * sc
  return jnp.maximum(y * sc + sh, 0.0)


def _layer_body(h3, a3, w1, b1, w2, b2, g, bb,
                out_ref, y_scr, st_scr, first):
  i = pl.program_id(0)
  r = i % GRID

  @pl.when(i == 0)
  def _():
    st_scr[...] = jnp.zeros((8, D), jnp.float32)

  @pl.when(i < GRID)
  def _():
    y = _mlp_y(h3, a3, w1, b1, w2, b2, first)
    y_scr[pl.ds(r * R, R), :] = y
    st_scr[...] += jnp.concatenate(
        [jnp.sum(y, axis=0)[None], jnp.sum(y * y, axis=0)[None],
         jnp.zeros((6, D), jnp.float32)], axis=0)

  @pl.when(i >= GRID)
  def _():
    h = _bn_h(y_scr[pl.ds(r * R, R), :], st_scr, g, bb)
    out_ref[0] = h[:, :DH]
    out_ref[1] = h[:, DH:]


def _layer4_body(h3, a3, w1, b1, w2, b2, g, bb, oh,
                 fw1, fb1, fw2, fb2, out_ref, y_scr, st_scr, ps_scr, pc_scr):
  i = pl.program_id(0)
  r = i % GRID

  @pl.when(i == 0)
  def _():
    st_scr[...] = jnp.zeros((8, D), jnp.float32)
    ps_scr[...] = jnp.zeros((G, D), jnp.float32)
    pc_scr[...] = jnp.zeros((8, G), jnp.float32)

  @pl.when(i < GRID)
  def _():
    y = _mlp_y(h3, a3, w1, b1, w2, b2, False)
    y_scr[pl.ds(r * R, R), :] = y
    st_scr[...] += jnp.concatenate(
        [jnp.sum(y, axis=0)[None], jnp.sum(y * y, axis=0)[None],
         jnp.zeros((6, D), jnp.float32)], axis=0)

  @pl.when(jnp.logical_and(i >= GRID, i < 2 * GRID))
  def _():
    h = _bn_h(y_scr[pl.ds(r * R, R), :], st_scr, g, bb)
    ohb = oh[0]
    ps_scr[...] += lax.dot_general(ohb, h, (((1,), (0,)), ((), ())),
                                   preferred_element_type=jnp.float32)
    pc_scr[...] += jnp.concatenate(
        [jnp.sum(ohb, axis=1)[None], jnp.zeros((7, G), jnp.float32)], axis=0)

  @pl.when(i == 2 * GRID)
  def _():
    cnt = jnp.maximum(pc_scr[0, :], 1.0)[:, None]
    pooled = ps_scr[...] / cnt
    rr = jnp.maximum(
        jnp.dot(pooled, fw1[...], preferred_element_type=jnp.float32)
        + fb1[...], 0.0)
    out_ref[...] = jnp.dot(rr, fw2[...],
                           preferred_element_type=jnp.float32) + fb2[...]


def _split_spec(w):
  def imap(i):
    return (0, jnp.where(i < GRID, i % GRID, 0), 0)
  return pl.BlockSpec((2, R, w), imap)


def _layer(hv, aggv, w1, b1, w2, b2, g, bb, first):
  wh = 16 if first else DH
  body = functools.partial(_layer_body, first=first)
  return pl.pallas_call(
      body,
      grid=(2 * GRID,),
      in_specs=[
          pl.BlockSpec((R, 16), lambda i: (jnp.where(i < GRID, i % GRID, 0),
                                           0)) if first else _split_spec(DH),
          _split_spec(wh),
          _full((1, D) if first else (D, D)), _full((1, D)),
          _full((D, D)), _full((1, D)),
          _full((1, D)), _full((1, D)),
      ],
      out_specs=pl.BlockSpec(
          (2, R, DH), lambda i: (0, jnp.where(i >= GRID, i % GRID, 0), 0)),
      out_shape=jax.ShapeDtypeStruct((2, N, DH), jnp.float32),
      scratch_shapes=[
          pltpu.VMEM((N, D), jnp.float32),
          pltpu.VMEM((8, D), jnp.float32),
      ],
  )(hv, aggv, w1, b1, w2, b2, g, bb)


def _layer4(hv, aggv, w1, b1, w2, b2, g, bb, oh, fw1, fb1, fw2, fb2):
  def oh_map(i):
    return (jnp.where(jnp.logical_and(i >= GRID, i < 2 * GRID),
                      i % GRID, 0), 0, 0)
  return pl.pallas_call(
      _layer4_body,
      grid=(2 * GRID + 1,),
      in_specs=[
          _split_spec(DH), _split_spec(DH),
          _full((D, D)), _full((1, D)), _full((D, D)), _full((1, D)),
          _full((1, D)), _full((1, D)),
          pl.BlockSpec((1, G, R), oh_map),
          _full((D, 128)), _full((1, 128)), _full((128, 10)), _full((1, 10)),
      ],
      out_specs=_full((G, 10)),
      out_shape=jax.ShapeDtypeStruct((G, 10), jnp.float32),
      scratch_shapes=[
          pltpu.VMEM((N, D), jnp.float32),
          pltpu.VMEM((8, D), jnp.float32),
          pltpu.VMEM((G, D), jnp.float32),
          pltpu.VMEM((8, G), jnp.float32),
      ],
  )(hv, aggv, w1, b1, w2, b2, g, bb, oh, fw1, fb1, fw2, fb2)


def kernel(x, edge_index, batch,
           w1_1, b1_1, w1_2, b1_2, bn1_g, bn1_b,
           w2_1, b2_1, w2_2, b2_2, bn2_g, bn2_b,
           w3_1, b3_1, w3_2, b3_2, bn3_g, bn3_b,
           w4_1, b4_1, w4_2, b4_2, bn4_g, bn4_b,
           fc1_w, fc1_b, fc2_w, fc2_b):
  src = edge_index[0]
  srcm = jnp.concatenate([src, src + N]).reshape(2 * NROWS, BE)
  dstm = edge_index[1].reshape(NROWS, BE)

  x16 = jnp.pad(x, ((0, 0), (0, 15)))
  agg1 = _sc_agg1(x16, srcm, dstm)
  hv = _layer(x16, agg1.reshape(2, N, 16),
              w1_1, b1_1.reshape(1, D), w1_2, b1_2.reshape(1, D),
              bn1_g.reshape(1, D), bn1_b.reshape(1, D), first=True)

  for (w1, b1, w2, b2, g, b) in (
      (w2_1, b2_1, w2_2, b2_2, bn2_g, bn2_b),
      (w3_1, b3_1, w3_2, b3_2, bn3_g, bn3_b)):
    aggf = _sc_agg(hv.reshape(2 * N, DH), srcm, dstm)
    hv = _layer(hv, aggf.reshape(2, N, DH),
                w1, b1.reshape(1, D), w2, b2.reshape(1, D),
                g.reshape(1, D), b.reshape(1, D), first=False)

  aggf = _sc_agg(hv.reshape(2 * N, DH), srcm, dstm)
  oh = (batch.reshape(GRID, 1, R)
        == jnp.arange(G, dtype=jnp.int32)[None, :, None]
        ).astype(jnp.float32)
  return _layer4(hv, aggf.reshape(2, N, DH),
                 w4_1, b4_1.reshape(1, D), w4_2, b4_2.reshape(1, D),
                 bn4_g.reshape(1, D), bn4_b.reshape(1, D), oh,
                 fc1_w, fc1_b.reshape(1, 128), fc2_w, fc2_b.reshape(1, 10))
